# baseline (speedup 1.0000x reference)
import functools

import jax
import jax.numpy as jnp
from jax import lax
from jax.experimental import pallas as pl
from jax.experimental.pallas import tpu as pltpu
from jax.experimental.pallas import tpu_sc as plsc

VOCAB = 100000
EMBED = 64
NFEAT = 26
HIDDEN = 1024
NCLASS = 2
BATCH = 4096

_NC = 2
_NS = 16
_NW = _NC * _NS

_NPAIR = NFEAT // 2
_BPW = BATCH // _NW


def _gather_sc(x2d, emb):
    mesh = plsc.VectorSubcoreMesh(core_axis_name="c", subcore_axis_name="s")

    @functools.partial(
        pl.kernel,
        mesh=mesh,
        compiler_params=pltpu.CompilerParams(use_tc_tiling_on_sc=False),
        out_type=jax.ShapeDtypeStruct((_NPAIR, BATCH, 2 * EMBED), jnp.float32),
        scratch_types=[
            pltpu.VMEM((_BPW,), jnp.int32),
            pltpu.VMEM((_BPW,), jnp.int32),
            pltpu.VMEM((_BPW, EMBED), jnp.float32),
            pltpu.VMEM((_BPW, EMBED), jnp.float32),
            pltpu.SemaphoreType.DMA,
        ],
    )
    def k(x_hbm, emb_hbm, out_hbm, idx_e, idx_o, rows_e, rows_o, sem):
        wid = lax.axis_index("s") * _NC + lax.axis_index("c")
        b0 = wid * _BPW
        for d in range(_NPAIR):
            pltpu.sync_copy(x_hbm.at[2 * d, pl.ds(b0, _BPW)], idx_e)
            pltpu.sync_copy(x_hbm.at[2 * d + 1, pl.ds(b0, _BPW)], idx_o)
            ce = pltpu.async_copy(emb_hbm.at[idx_e], rows_e, sem)
            co = pltpu.async_copy(emb_hbm.at[idx_o], rows_o, sem)
            ce.wait()
            co.wait()
            pltpu.sync_copy(
                rows_e, out_hbm.at[d, pl.ds(b0, _BPW), pl.ds(0, EMBED)])
            pltpu.sync_copy(
                rows_o, out_hbm.at[d, pl.ds(b0, _BPW), pl.ds(EMBED, EMBED)])

    return k(x2d, emb)


_BB = 512


def _mlp_body(a_ref, w1_ref, b1_ref, w2_ref, b2_ref, o_ref):
    acc = jnp.zeros((_BB, HIDDEN), jnp.float32)
    for d in range(_NPAIR):
        acc += lax.dot_general(
            a_ref[d], w1_ref[:, d * 2 * EMBED:(d + 1) * 2 * EMBED],
            (((1,), (1,)), ((), ())), preferred_element_type=jnp.float32)
    h = jnp.maximum(acc + b1_ref[...], 0.0)
    o = lax.dot_general(h, w2_ref[...], (((1,), (1,)), ((), ())),
                        preferred_element_type=jnp.float32)
    o_ref[...] = o + b2_ref[...]


def _mlp_tc(a3, W1, b1, W2, b2):
    din = NFEAT * EMBED
    return pl.pallas_call(
        _mlp_body,
        grid=(BATCH // _BB,),
        in_specs=[
            pl.BlockSpec((_NPAIR, _BB, 2 * EMBED), lambda i: (0, i, 0)),
            pl.BlockSpec((HIDDEN, din), lambda i: (0, 0)),
            pl.BlockSpec((1, HIDDEN), lambda i: (0, 0)),
            pl.BlockSpec((NCLASS, HIDDEN), lambda i: (0, 0)),
            pl.BlockSpec((1, NCLASS), lambda i: (0, 0)),
        ],
        out_specs=pl.BlockSpec((_BB, NCLASS), lambda i: (i, 0)),
        out_shape=jax.ShapeDtypeStruct((BATCH, NCLASS), jnp.float32),
    )(a3, W1, b1.reshape(1, HIDDEN), W2, b2.reshape(1, NCLASS))


def kernel(x, emb, W1, b1, W2, b2):
    a3 = _gather_sc(x.astype(jnp.int32).T, emb)
    return _mlp_tc(a3, W1, b1, W2, b2)


# speedup vs baseline: 1.0252x; 1.0252x over previous
import functools

import jax
import jax.numpy as jnp
from jax import lax
from jax.experimental import pallas as pl
from jax.experimental.pallas import tpu as pltpu
from jax.experimental.pallas import tpu_sc as plsc

VOCAB = 100000
EMBED = 64
NFEAT = 26
HIDDEN = 1024
NCLASS = 2
BATCH = 4096

_NC = 2
_NS = 16
_NW = _NC * _NS

_ROWS = BATCH * NFEAT
_BPW = BATCH // _NW
_BPC = 32
_NCHUNK = _BPW // _BPC
_CROWS = _BPC * NFEAT


def _gather_sc(x2d, emb):
    mesh = plsc.VectorSubcoreMesh(core_axis_name="c", subcore_axis_name="s")

    @functools.partial(
        pl.kernel,
        mesh=mesh,
        compiler_params=pltpu.CompilerParams(use_tc_tiling_on_sc=False),
        out_type=jax.ShapeDtypeStruct((_ROWS, EMBED), jnp.float32),
        scratch_types=[
            pltpu.VMEM((_BPW, NFEAT), jnp.int32),
            pltpu.VMEM((_CROWS, EMBED), jnp.float32),
            pltpu.VMEM((_CROWS, EMBED), jnp.float32),
            pltpu.SemaphoreType.DMA,
            pltpu.SemaphoreType.DMA,
        ],
    )
    def k(x_hbm, emb_hbm, out_hbm, idx_v, rows_a, rows_b, sem_a, sem_b):
        wid = lax.axis_index("s") * _NC + lax.axis_index("c")
        b0 = wid * _BPW
        bufs = (rows_a, rows_b)
        sems = (sem_a, sem_b)
        pltpu.sync_copy(x_hbm.at[pl.ds(b0, _BPW)], idx_v)

        def fire(c):
            buf, sem = bufs[c % 2], sems[c % 2]

            def body(i, carry):
                pltpu.async_copy(
                    emb_hbm.at[idx_v.at[c * _BPC + i]],
                    buf.at[pl.ds(i * NFEAT, NFEAT)], sem)
                return carry

            lax.fori_loop(0, _BPC, body, 0)

        def drain(c):
            pltpu.make_async_copy(
                emb_hbm.at[pl.ds(0, _CROWS)], bufs[c % 2], sems[c % 2]).wait()

        fire(0)
        fire(1)
        for c in range(_NCHUNK):
            drain(c)
            pltpu.sync_copy(
                bufs[c % 2],
                out_hbm.at[pl.ds(b0 * NFEAT + c * _CROWS, _CROWS)])
            if c + 2 < _NCHUNK:
                fire(c + 2)

    return k(x2d, emb)


_BB = 512


def _mlp_body(a_ref, w1_ref, b1_ref, w2_ref, b2_ref, o_ref):
    h = lax.dot_general(a_ref[...], w1_ref[...], (((1,), (1,)), ((), ())),
                        preferred_element_type=jnp.float32)
    h = jnp.maximum(h + b1_ref[...], 0.0)
    o = lax.dot_general(h, w2_ref[...], (((1,), (1,)), ((), ())),
                        preferred_element_type=jnp.float32)
    o_ref[...] = o + b2_ref[...]


def _mlp_tc(a, W1, b1, W2, b2):
    din = NFEAT * EMBED
    return pl.pallas_call(
        _mlp_body,
        grid=(BATCH // _BB,),
        in_specs=[
            pl.BlockSpec((_BB, din), lambda i: (i, 0)),
            pl.BlockSpec((HIDDEN, din), lambda i: (0, 0)),
            pl.BlockSpec((1, HIDDEN), lambda i: (0, 0)),
            pl.BlockSpec((NCLASS, HIDDEN), lambda i: (0, 0)),
            pl.BlockSpec((1, NCLASS), lambda i: (0, 0)),
        ],
        out_specs=pl.BlockSpec((_BB, NCLASS), lambda i: (i, 0)),
        out_shape=jax.ShapeDtypeStruct((BATCH, NCLASS), jnp.float32),
    )(a, W1, b1.reshape(1, HIDDEN), W2, b2.reshape(1, NCLASS))


def kernel(x, emb, W1, b1, W2, b2):
    gathered = _gather_sc(x.astype(jnp.int32), emb)
    a = gathered.reshape(BATCH, NFEAT * EMBED)
    return _mlp_tc(a, W1, b1, W2, b2)
